# Initial kernel scaffold; baseline (speedup 1.0000x reference)
import jax
import jax.numpy as jnp
from jax.experimental import pallas as pl


def kernel(h, edge_index, e, emb, eps, mlp_W, mlp_b, bn_scale, bn_bias, lin_W, lin_b, pred_W, pred_b):
    raise NotImplementedError("write your pallas kernel here")


# speedup vs baseline: 1.0583x; 1.0583x over previous
import functools

import jax
import jax.numpy as jnp
from jax.experimental import pallas as pl
from jax.experimental.pallas import tpu as pltpu

_N = 10000
_D = 128


def _embed_body(h_ref, emb_ref, x_ref, cs_ref):
    h = h_ref[...]
    cols = jax.lax.broadcasted_iota(jnp.int32, (_N, _D), 1)
    oh = (cols == h).astype(jnp.float32)
    x = jnp.dot(oh, emb_ref[...], preferred_element_type=jnp.float32)
    x_ref[...] = x
    cs_ref[...] = jnp.sum(x, axis=0, keepdims=True)


def _dense_body(eps_ref, x_ref, a_ref, W_ref, b_ref, sc_ref, bi_ref, lW_ref,
                lb_ref, out_ref, cs_ref):
    x = x_ref[...]
    y0 = (1.0 + eps_ref[0]) * x + a_ref[...]
    t = jnp.dot(y0, W_ref[...], preferred_element_type=jnp.float32) + b_ref[...]
    mu = jnp.mean(t, axis=0, keepdims=True)
    d = t - mu
    var = jnp.mean(d * d, axis=0, keepdims=True)
    y = d * jax.lax.rsqrt(var + 1e-5) * sc_ref[...] + bi_ref[...]
    y = jnp.maximum(y, 0.0)
    xr = x + y
    out = jnp.dot(xr, lW_ref[...], preferred_element_type=jnp.float32) + lb_ref[...]
    out_ref[...] = out
    cs_ref[...] = jnp.sum(out, axis=0, keepdims=True)


_embed_call = pl.pallas_call(
    _embed_body,
    out_shape=(jax.ShapeDtypeStruct((_N, _D), jnp.float32),
               jax.ShapeDtypeStruct((1, _D), jnp.float32)),
)

_dense_call = pl.pallas_call(
    _dense_body,
    in_specs=[pl.BlockSpec(memory_space=pltpu.SMEM)] + [pl.BlockSpec()] * 8,
    out_shape=(jax.ShapeDtypeStruct((_N, _D), jnp.float32),
               jax.ShapeDtypeStruct((1, _D), jnp.float32)),
)


def kernel(h, edge_index, e, emb, eps, mlp_W, mlp_b, bn_scale, bn_bias,
           lin_W, lin_b, pred_W, pred_b):
    del e
    src = edge_index[0]
    dst = edge_index[1]
    emb_pad = jnp.zeros((_D, _D), jnp.float32).at[:emb.shape[0]].set(emb)

    x, cs0 = _embed_call(h.reshape(_N, 1).astype(jnp.int32), emb_pad)
    colsums = [cs0]
    for i in range(4):
        aggr = jax.ops.segment_sum(jnp.take(x, src, axis=0), dst,
                                   num_segments=_N)
        x, cs = _dense_call(eps[i].reshape(1), x, aggr, mlp_W[i],
                            mlp_b[i].reshape(1, _D), bn_scale[i].reshape(1, _D),
                            bn_bias[i].reshape(1, _D), lin_W[i],
                            lin_b[i].reshape(1, _D))
        colsums.append(cs)

    score = jnp.zeros((1, 1), jnp.float32)
    for i, cs in enumerate(colsums):
        score = score + jnp.dot(cs / _N, pred_W[i])
    score = score.reshape(1) + jnp.sum(pred_b, axis=0)
    return score


# speedup vs baseline: 6.7785x; 6.4051x over previous
import functools

import jax
import jax.numpy as jnp
from jax import lax
from jax.experimental import pallas as pl
from jax.experimental.pallas import tpu as pltpu
from jax.experimental.pallas import tpu_sc as plsc

_N = 10000
_E = 320000
_D = 128
_NW = 16
_C = 80
_NCH = _E // (_NW * _C)
_NH = 5120
_AP = 5248
_RPT = _AP // 16


def _embed_body(h_ref, emb_ref, x_ref, cs_ref):
    h = h_ref[...]
    cols = jax.lax.broadcasted_iota(jnp.int32, (_N, _D), 1)
    oh = (cols == h).astype(jnp.float32)
    x = jnp.dot(oh, emb_ref[...], preferred_element_type=jnp.float32)
    x_ref[...] = x
    cs_ref[...] = jnp.sum(x, axis=0, keepdims=True)


def _sc_aggr_body(x_hbm, src_hbm, dst_hbm, out_hbm,
                  src_v, dst_v, rows_v, acc, sem0, sem1):
    c = lax.axis_index("c")
    s = lax.axis_index("s")
    wid = s
    base = c * _NH

    zbuf = rows_v.at[0]

    def zrow(r, carry):
        for q in range(_D // 16):
            zbuf[r, pl.ds(16 * q, 16)] = jnp.zeros((16,), jnp.float32)
        return carry

    lax.fori_loop(0, _C, zrow, 0)
    for t in range(_RPT // _C):
        pltpu.sync_copy(zbuf, acc.at[pl.ds(s * _RPT + t * _C, _C)])
    pltpu.sync_copy(zbuf.at[pl.ds(0, _RPT % _C)],
                    acc.at[pl.ds(s * _RPT + (_RPT // _C) * _C, _RPT % _C)])

    pltpu.sync_copy(src_hbm.at[wid], src_v)
    pltpu.sync_copy(dst_hbm.at[wid], dst_v)

    junk = _NH + s

    def fixrow(r, carry):
        for q in range(_C // 16):
            v = dst_v[r, pl.ds(16 * q, 16)] - base
            ok = (v >= 0) & (v < _NH)
            dst_v[r, pl.ds(16 * q, 16)] = jnp.where(ok, v, junk)
        return carry

    lax.fori_loop(0, _NCH, fixrow, 0)
    plsc.subcore_barrier()

    pltpu.async_copy(x_hbm.at[src_v.at[0]], rows_v.at[0], sem0)
    pltpu.async_copy(x_hbm.at[src_v.at[1]], rows_v.at[1], sem1)

    def step(i, carry):
        k = 2 * i
        pltpu.make_async_copy(x_hbm.at[src_v.at[k]], rows_v.at[0], sem0).wait()
        pltpu.sync_copy(rows_v.at[0], acc.at[dst_v.at[k]], add=True)
        pltpu.async_copy(x_hbm.at[src_v.at[k + 2]], rows_v.at[0], sem0)
        pltpu.make_async_copy(x_hbm.at[src_v.at[k + 1]], rows_v.at[1], sem1).wait()
        pltpu.sync_copy(rows_v.at[1], acc.at[dst_v.at[k + 1]], add=True)
        pltpu.async_copy(x_hbm.at[src_v.at[k + 3]], rows_v.at[1], sem1)
        return carry

    lax.fori_loop(0, _NCH // 2 - 1, step, 0)
    k = _NCH - 2
    pltpu.make_async_copy(x_hbm.at[src_v.at[k]], rows_v.at[0], sem0).wait()
    pltpu.sync_copy(rows_v.at[0], acc.at[dst_v.at[k]], add=True)
    pltpu.make_async_copy(x_hbm.at[src_v.at[k + 1]], rows_v.at[1], sem1).wait()
    pltpu.sync_copy(rows_v.at[1], acc.at[dst_v.at[k + 1]], add=True)

    plsc.subcore_barrier()
    pltpu.sync_copy(acc.at[pl.ds(s * _RPT, _RPT)],
                    out_hbm.at[c].at[pl.ds(s * _RPT, _RPT)])


_sc_aggr = functools.partial(
    pl.kernel,
    mesh=plsc.VectorSubcoreMesh(core_axis_name="c", subcore_axis_name="s"),
    out_type=jax.ShapeDtypeStruct((2, _AP, _D), jnp.float32),
    scratch_types=[
        pltpu.VMEM((_NCH, _C), jnp.int32),
        pltpu.VMEM((_NCH, _C), jnp.int32),
        pltpu.VMEM((2, _C, _D), jnp.float32),
        pltpu.VMEM_SHARED((_AP, _D), jnp.float32),
        pltpu.SemaphoreType.DMA,
        pltpu.SemaphoreType.DMA,
    ],
)(_sc_aggr_body)


def _dense_body(eps_ref, x_ref, a_ref, W_ref, b_ref, sc_ref, bi_ref, lW_ref,
                lb_ref, out_ref, cs_ref):
    x = x_ref[...]
    a = jnp.concatenate([a_ref[0, :_NH], a_ref[1, : _N - _NH]], axis=0)
    y0 = (1.0 + eps_ref[0]) * x + a
    t = jnp.dot(y0, W_ref[...], preferred_element_type=jnp.float32) + b_ref[...]
    mu = jnp.mean(t, axis=0, keepdims=True)
    d = t - mu
    var = jnp.mean(d * d, axis=0, keepdims=True)
    y = d * jax.lax.rsqrt(var + 1e-5) * sc_ref[...] + bi_ref[...]
    y = jnp.maximum(y, 0.0)
    xr = x + y
    out = jnp.dot(xr, lW_ref[...], preferred_element_type=jnp.float32) + lb_ref[...]
    out_ref[...] = out
    cs_ref[...] = jnp.sum(out, axis=0, keepdims=True)


_embed_call = pl.pallas_call(
    _embed_body,
    out_shape=(jax.ShapeDtypeStruct((_N, _D), jnp.float32),
               jax.ShapeDtypeStruct((1, _D), jnp.float32)),
)

_dense_call = pl.pallas_call(
    _dense_body,
    in_specs=[pl.BlockSpec(memory_space=pltpu.SMEM)] + [pl.BlockSpec()] * 8,
    out_shape=(jax.ShapeDtypeStruct((_N, _D), jnp.float32),
               jax.ShapeDtypeStruct((1, _D), jnp.float32)),
)


def kernel(h, edge_index, e, emb, eps, mlp_W, mlp_b, bn_scale, bn_bias,
           lin_W, lin_b, pred_W, pred_b):
    del e
    src = edge_index[0]
    dst = edge_index[1]
    emb_pad = jnp.zeros((_D, _D), jnp.float32).at[:emb.shape[0]].set(emb)

    src3 = src.reshape(_NW, _NCH, _C).astype(jnp.int32)
    dst3 = dst.reshape(_NW, _NCH, _C).astype(jnp.int32)
    x, cs0 = _embed_call(h.reshape(_N, 1).astype(jnp.int32), emb_pad)

    def layer(x, p):
        Wi, bi, sci, bii, lWi, lbi, epsi = p
        aggr = _sc_aggr(x, src3, dst3)
        x, cs = _dense_call(epsi.reshape(1), x, aggr, Wi,
                            bi.reshape(1, _D), sci.reshape(1, _D),
                            bii.reshape(1, _D), lWi, lbi.reshape(1, _D))
        return x, cs

    x, css = lax.scan(layer, x,
                      (mlp_W, mlp_b, bn_scale, bn_bias, lin_W, lin_b, eps))

    colsums = [cs0] + [css[i] for i in range(4)]
    score = jnp.zeros((1, 1), jnp.float32)
    for i, cs in enumerate(colsums):
        score = score + jnp.dot(cs / _N, pred_W[i])
    score = score.reshape(1) + jnp.sum(pred_b, axis=0)
    return score
